# baseline (speedup 1.0000x reference)
import functools

import jax
import jax.numpy as jnp
from jax import lax
from jax.experimental import pallas as pl
from jax.experimental.pallas import tpu as pltpu
from jax.experimental.pallas import tpu_sc as plsc

N = 10000
EH = 160000
E = 2 * EH
D = 128

NC = 2
NS = 16
NW = NC * NS
CH = 128
NCHUNK = E // CH
CPW = 80
NCPAD = NW * CPW
NBUF = 4
SNBUF = 2
NP = 10240
RPT = NP // NS

_mesh = plsc.VectorSubcoreMesh(core_axis_name="c", subcore_axis_name="s")


def _worker_span():
    c = lax.axis_index("c")
    s = lax.axis_index("s")
    wid = s * NC + c
    base = wid * CPW
    cnt = jnp.clip(NCHUNK - base, 0, CPW)
    return base, cnt


def _sc_scatter_add(h2, idx2, zeros):
    @functools.partial(
        pl.kernel,
        out_type=jax.ShapeDtypeStruct((NC, NP, D), jnp.float32),
        mesh=_mesh,
        scratch_types=[
            pltpu.VMEM((CPW, CH), jnp.int32),
            pltpu.VMEM((SNBUF, CH, D), jnp.float32),
            pltpu.VMEM_SHARED((NP, D), jnp.float32),
        ] + [pltpu.SemaphoreType.DMA] * SNBUF,
    )
    def k(h_hbm, idx_hbm, z_hbm, out_hbm, idx_v, rows_v, acc, *sems):
        s = lax.axis_index("s")
        base, cnt = _worker_span()
        pltpu.sync_copy(z_hbm.at[pl.ds(s * RPT, RPT)], acc.at[pl.ds(s * RPT, RPT)])
        pltpu.sync_copy(idx_hbm.at[pl.ds(base, CPW)], idx_v)
        plsc.subcore_barrier()

        def load(i, b):
            return pltpu.make_async_copy(
                h_hbm.at[pl.ds((base + i) * CH, CH)], rows_v.at[b], sems[b])

        def drain(b):
            pltpu.make_async_copy(
                h_hbm.at[pl.ds(0, CH)], rows_v.at[b], sems[b]).wait()

        for b in range(SNBUF):
            @pl.when(b < cnt)
            def _(b=b):
                load(b, b).start()

        @pl.loop(0, CPW, step=SNBUF)
        def _(i0):
            for b in range(SNBUF):
                i = i0 + b

                @pl.when(i < cnt)
                def _(i=i, b=b):
                    drain(b)
                    pltpu.sync_copy(rows_v.at[b], acc.at[idx_v.at[i]], add=True)

                    @pl.when(i + SNBUF < cnt)
                    def _():
                        load(i + SNBUF, b).start()

        plsc.subcore_barrier()
        c = lax.axis_index("c")
        pltpu.sync_copy(acc.at[pl.ds(s * RPT, RPT)],
                        out_hbm.at[c, pl.ds(s * RPT, RPT)])

    return k(h2.reshape(E, D), idx2, zeros)


def _sc_gather(table, idx2):
    @functools.partial(
        pl.kernel,
        out_type=jax.ShapeDtypeStruct((E, D), jnp.float32),
        mesh=_mesh,
        scratch_types=[
            pltpu.VMEM((CPW, CH), jnp.int32),
            pltpu.VMEM((NBUF, CH, D), jnp.float32),
        ] + [pltpu.SemaphoreType.DMA] * NBUF,
    )
    def k(t_hbm, idx_hbm, out_hbm, idx_v, rows_v, *sems):
        base, cnt = _worker_span()
        pltpu.sync_copy(idx_hbm.at[pl.ds(base, CPW)], idx_v)

        def gat(i, b):
            return pltpu.make_async_copy(
                t_hbm.at[idx_v.at[i]], rows_v.at[b], sems[b])

        def drain(b):
            pltpu.make_async_copy(
                t_hbm.at[pl.ds(0, CH)], rows_v.at[b], sems[b]).wait()

        for b in range(NBUF):
            @pl.when(b < cnt)
            def _(b=b):
                gat(b, b).start()

        @pl.loop(0, CPW, step=NBUF)
        def _(i0):
            for b in range(NBUF):
                i = i0 + b

                @pl.when(i < cnt)
                def _(i=i, b=b):
                    drain(b)
                    pltpu.sync_copy(rows_v.at[b],
                                    out_hbm.at[pl.ds((base + i) * CH, CH)])

                    @pl.when(i + NBUF < cnt)
                    def _():
                        gat(i + NBUF, b).start()

    return k(table, idx2)


def _matmul_body(x_ref, w_ref, o_ref):
    o_ref[...] = jnp.dot(x_ref[...], w_ref[...],
                         preferred_element_type=jnp.float32)


def _tc_matmul(x, w, bm=2000):
    m, kdim = x.shape
    dout = w.shape[1]
    return pl.pallas_call(
        _matmul_body,
        grid=(m // bm,),
        in_specs=[
            pl.BlockSpec((bm, kdim), lambda i: (i, 0)),
            pl.BlockSpec((kdim, dout), lambda i: (0, 0)),
        ],
        out_specs=pl.BlockSpec((bm, dout), lambda i: (i, 0)),
        out_shape=jax.ShapeDtypeStruct((m, dout), jnp.float32),
    )(x, w)


def _lin1_body(ea_ref, g_ref, w_ref, b_ref, o_ref):
    o_ref[...] = jnp.maximum(
        g_ref[...]
        + jnp.dot(ea_ref[...], w_ref[...], preferred_element_type=jnp.float32)
        + b_ref[...], 0.0)


def _tc_lin1(ea, gq, w1e, b1, bp=2000):
    de = ea.shape[-1]
    return pl.pallas_call(
        _lin1_body,
        grid=(E // bp,),
        in_specs=[
            pl.BlockSpec((bp, de), lambda i: (i, 0)),
            pl.BlockSpec((bp, D), lambda i: (i, 0)),
            pl.BlockSpec((de, D), lambda i: (0, 0)),
            pl.BlockSpec((1, D), lambda i: (0, 0)),
        ],
        out_specs=pl.BlockSpec((bp, D), lambda i: (i, 0)),
        out_shape=jax.ShapeDtypeStruct((E, D), jnp.float32),
    )(ea, gq, w1e, b1)


def _combine_body(h_ref, g_ref, w_ref, b_ref, o_ref):
    w = w_ref[...]
    b = b_ref[...]
    hb = h_ref[...]
    gb = g_ref[...]
    he, ho = hb[:, :D], hb[:, D:]
    ze = jnp.dot(gb[:, :D] - he, w, preferred_element_type=jnp.float32)
    zo = jnp.dot(gb[:, D:] - ho, w, preferred_element_type=jnp.float32)
    ne = jnp.maximum(he + zo + b, 0.0)
    no = jnp.maximum(ho + ze + b, 0.0)
    o_ref[...] = jnp.concatenate([ne, no], axis=1)


def _tc_combine(h2, g2, w, b, bp=1000):
    return pl.pallas_call(
        _combine_body,
        grid=(EH // bp,),
        in_specs=[
            pl.BlockSpec((bp, 2 * D), lambda i: (i, 0)),
            pl.BlockSpec((bp, 2 * D), lambda i: (i, 0)),
            pl.BlockSpec((D, D), lambda i: (0, 0)),
            pl.BlockSpec((1, D), lambda i: (0, 0)),
        ],
        out_specs=pl.BlockSpec((bp, 2 * D), lambda i: (i, 0)),
        out_shape=jax.ShapeDtypeStruct((EH, 2 * D), jnp.float32),
    )(h2.reshape(EH, 2 * D), g2.reshape(EH, 2 * D), w, b)


def _sum2_body(p_ref, o_ref):
    o_ref[...] = p_ref[0] + p_ref[1]


def _tc_sum2(parts, bn=2000):
    return pl.pallas_call(
        _sum2_body,
        grid=(N // bn,),
        in_specs=[pl.BlockSpec((2, bn, D), lambda i: (0, i, 0))],
        out_specs=pl.BlockSpec((bn, D), lambda i: (i, 0)),
        out_shape=jax.ShapeDtypeStruct((N, D), jnp.float32),
    )(parts)


def _final_body(x_ref, p_ref, wx_ref, wh_ref, b_ref, o_ref):
    agg = p_ref[0] + p_ref[1]
    o_ref[...] = jnp.maximum(
        jnp.dot(x_ref[...], wx_ref[...], preferred_element_type=jnp.float32)
        + jnp.dot(agg, wh_ref[...], preferred_element_type=jnp.float32)
        + b_ref[...], 0.0)


def _tc_final(x, parts, wax, wah, ba, bn=2000):
    return pl.pallas_call(
        _final_body,
        grid=(N // bn,),
        in_specs=[
            pl.BlockSpec((bn, D), lambda i: (i, 0)),
            pl.BlockSpec((2, bn, D), lambda i: (0, i, 0)),
            pl.BlockSpec((D, D), lambda i: (0, 0)),
            pl.BlockSpec((D, D), lambda i: (0, 0)),
            pl.BlockSpec((1, D), lambda i: (0, 0)),
        ],
        out_specs=pl.BlockSpec((bn, D), lambda i: (i, 0)),
        out_shape=jax.ShapeDtypeStruct((N, D), jnp.float32),
    )(x, parts, wax, wah, ba)


def kernel(x, edge_attr, W1, b1, Wm1, bm1, Wm2, bm2, Wm3, bm3, Wa, ba,
           edge_index):
    pad = jnp.zeros((NCPAD * CH - E,), jnp.int32)
    srcp = jnp.concatenate([edge_index[0].astype(jnp.int32), pad]).reshape(NCPAD, CH)
    dstp = jnp.concatenate([edge_index[1].astype(jnp.int32), pad]).reshape(NCPAD, CH)
    zeros = jnp.zeros((NP, D), jnp.float32)

    q = _tc_matmul(x, W1[:D])
    gq = _sc_gather(q, srcp)
    h = _tc_lin1(edge_attr, gq, W1[D:], b1.reshape(1, D))

    for w, b in ((Wm1, bm1), (Wm2, bm2), (Wm3, bm3)):
        parts = _sc_scatter_add(h, dstp, zeros)
        agg = _tc_sum2(parts)
        gd = _sc_gather(agg, dstp)
        h = _tc_combine(h, gd, w, b.reshape(1, D))

    parts = _sc_scatter_add(h, dstp, zeros)
    return _tc_final(x, parts, Wa[:D], Wa[D:], ba.reshape(1, D))


# speedup vs baseline: 1.5729x; 1.5729x over previous
import functools

import jax
import jax.numpy as jnp
from jax import lax
from jax.experimental import pallas as pl
from jax.experimental.pallas import tpu as pltpu
from jax.experimental.pallas import tpu_sc as plsc

N = 10000
EH = 160000
E = 2 * EH
D = 128

NC = 2
NS = 16
NW = NC * NS
CH = 128
NCHUNK = E // CH
CPW = 80
NCPAD = NW * CPW
NBUF = 4
SNBUF = 2
NP = 10240
RPT = NP // NS

_mesh = plsc.VectorSubcoreMesh(core_axis_name="c", subcore_axis_name="s")


def _worker_span():
    c = lax.axis_index("c")
    s = lax.axis_index("s")
    wid = s * NC + c
    base = wid * CPW
    cnt = jnp.clip(NCHUNK - base, 0, CPW)
    return base, cnt


def _sc_scatter_add(h2, idx2, zeros):
    @functools.partial(
        pl.kernel,
        out_type=jax.ShapeDtypeStruct((NC, NP, D), jnp.float32),
        mesh=_mesh,
        scratch_types=[
            pltpu.VMEM((CPW, CH), jnp.int32),
            pltpu.VMEM((SNBUF, CH, D), jnp.float32),
            pltpu.VMEM_SHARED((NP, D), jnp.float32),
        ] + [pltpu.SemaphoreType.DMA] * SNBUF,
    )
    def k(h_hbm, idx_hbm, z_hbm, out_hbm, idx_v, rows_v, acc, *sems):
        s = lax.axis_index("s")
        base, cnt = _worker_span()
        pltpu.sync_copy(z_hbm.at[pl.ds(s * RPT, RPT)], acc.at[pl.ds(s * RPT, RPT)])
        pltpu.sync_copy(idx_hbm.at[pl.ds(base, CPW)], idx_v)
        plsc.subcore_barrier()

        def load(i, b):
            return pltpu.make_async_copy(
                h_hbm.at[pl.ds((base + i) * CH, CH)], rows_v.at[b], sems[b])

        def drain(b):
            pltpu.make_async_copy(
                h_hbm.at[pl.ds(0, CH)], rows_v.at[b], sems[b]).wait()

        for b in range(SNBUF):
            @pl.when(b < cnt)
            def _(b=b):
                load(b, b).start()

        @pl.loop(0, CPW, step=SNBUF)
        def _(i0):
            for b in range(SNBUF):
                i = i0 + b

                @pl.when(i < cnt)
                def _(i=i, b=b):
                    drain(b)
                    pltpu.sync_copy(rows_v.at[b], acc.at[idx_v.at[i]], add=True)

                    @pl.when(i + SNBUF < cnt)
                    def _():
                        load(i + SNBUF, b).start()

        plsc.subcore_barrier()
        c = lax.axis_index("c")
        pltpu.sync_copy(acc.at[pl.ds(s * RPT, RPT)],
                        out_hbm.at[c, pl.ds(s * RPT, RPT)])

    return k(h2.reshape(E, D), idx2, zeros)


def _sc_gather(table, idx2):
    @functools.partial(
        pl.kernel,
        out_type=jax.ShapeDtypeStruct((E, D), jnp.float32),
        mesh=_mesh,
        scratch_types=[
            pltpu.VMEM((CPW, CH), jnp.int32),
            pltpu.VMEM((NBUF, CH, D), jnp.float32),
        ] + [pltpu.SemaphoreType.DMA] * NBUF,
    )
    def k(t_hbm, idx_hbm, out_hbm, idx_v, rows_v, *sems):
        base, cnt = _worker_span()
        pltpu.sync_copy(idx_hbm.at[pl.ds(base, CPW)], idx_v)

        def gat(i, b):
            return pltpu.make_async_copy(
                t_hbm.at[idx_v.at[i]], rows_v.at[b], sems[b])

        def drain(b):
            pltpu.make_async_copy(
                t_hbm.at[pl.ds(0, CH)], rows_v.at[b], sems[b]).wait()

        for b in range(NBUF):
            @pl.when(b < cnt)
            def _(b=b):
                gat(b, b).start()

        @pl.loop(0, CPW, step=NBUF)
        def _(i0):
            for b in range(NBUF):
                i = i0 + b

                @pl.when(i < cnt)
                def _(i=i, b=b):
                    drain(b)
                    pltpu.sync_copy(rows_v.at[b],
                                    out_hbm.at[pl.ds((base + i) * CH, CH)])

                    @pl.when(i + NBUF < cnt)
                    def _():
                        gat(i + NBUF, b).start()

    return k(table, idx2)


def _matmul_body(x_ref, w_ref, o_ref):
    o_ref[...] = jnp.dot(x_ref[...], w_ref[...],
                         preferred_element_type=jnp.float32)


def _tc_matmul(x, w, bm=2000):
    m, kdim = x.shape
    dout = w.shape[1]
    return pl.pallas_call(
        _matmul_body,
        grid=(m // bm,),
        in_specs=[
            pl.BlockSpec((bm, kdim), lambda i: (i, 0)),
            pl.BlockSpec((kdim, dout), lambda i: (0, 0)),
        ],
        out_specs=pl.BlockSpec((bm, dout), lambda i: (i, 0)),
        out_shape=jax.ShapeDtypeStruct((m, dout), jnp.float32),
    )(x, w)


def _lin1_body(ea_ref, g_ref, w_ref, b_ref, o_ref):
    o_ref[...] = jnp.maximum(
        g_ref[...]
        + jnp.dot(ea_ref[...], w_ref[...], preferred_element_type=jnp.float32)
        + b_ref[...], 0.0)


def _tc_lin1(ea, gq, w1e, b1, bp=2000):
    de = ea.shape[-1]
    return pl.pallas_call(
        _lin1_body,
        grid=(E // bp,),
        in_specs=[
            pl.BlockSpec((bp, de), lambda i: (i, 0)),
            pl.BlockSpec((bp, D), lambda i: (i, 0)),
            pl.BlockSpec((de, D), lambda i: (0, 0)),
            pl.BlockSpec((1, D), lambda i: (0, 0)),
        ],
        out_specs=pl.BlockSpec((bp, D), lambda i: (i, 0)),
        out_shape=jax.ShapeDtypeStruct((E, D), jnp.float32),
    )(ea, gq, w1e, b1)


def _combine_body(h_ref, g_ref, w_ref, b_ref, o_ref):
    hb = h_ref[...]
    z = jnp.dot(g_ref[...] - hb, w_ref[...],
                preferred_element_type=jnp.float32)
    even = (lax.broadcasted_iota(jnp.int32, z.shape, 0) % 2) == 0
    zsw = jnp.where(even, jnp.roll(z, -1, axis=0), jnp.roll(z, 1, axis=0))
    o_ref[...] = jnp.maximum(hb + zsw + b_ref[...], 0.0)


def _tc_combine(h2, g2, w, b, bp=2000):
    return pl.pallas_call(
        _combine_body,
        grid=(E // bp,),
        in_specs=[
            pl.BlockSpec((bp, D), lambda i: (i, 0)),
            pl.BlockSpec((bp, D), lambda i: (i, 0)),
            pl.BlockSpec((D, D), lambda i: (0, 0)),
            pl.BlockSpec((1, D), lambda i: (0, 0)),
        ],
        out_specs=pl.BlockSpec((bp, D), lambda i: (i, 0)),
        out_shape=jax.ShapeDtypeStruct((E, D), jnp.float32),
    )(h2, g2, w, b)


def _sum2_body(p_ref, o_ref):
    o_ref[...] = p_ref[0] + p_ref[1]


def _tc_sum2(parts, bn=2000):
    return pl.pallas_call(
        _sum2_body,
        grid=(N // bn,),
        in_specs=[pl.BlockSpec((2, bn, D), lambda i: (0, i, 0))],
        out_specs=pl.BlockSpec((bn, D), lambda i: (i, 0)),
        out_shape=jax.ShapeDtypeStruct((N, D), jnp.float32),
    )(parts)


def _final_body(x_ref, p_ref, wx_ref, wh_ref, b_ref, o_ref):
    agg = p_ref[0] + p_ref[1]
    o_ref[...] = jnp.maximum(
        jnp.dot(x_ref[...], wx_ref[...], preferred_element_type=jnp.float32)
        + jnp.dot(agg, wh_ref[...], preferred_element_type=jnp.float32)
        + b_ref[...], 0.0)


def _tc_final(x, parts, wax, wah, ba, bn=2000):
    return pl.pallas_call(
        _final_body,
        grid=(N // bn,),
        in_specs=[
            pl.BlockSpec((bn, D), lambda i: (i, 0)),
            pl.BlockSpec((2, bn, D), lambda i: (0, i, 0)),
            pl.BlockSpec((D, D), lambda i: (0, 0)),
            pl.BlockSpec((D, D), lambda i: (0, 0)),
            pl.BlockSpec((1, D), lambda i: (0, 0)),
        ],
        out_specs=pl.BlockSpec((bn, D), lambda i: (i, 0)),
        out_shape=jax.ShapeDtypeStruct((N, D), jnp.float32),
    )(x, parts, wax, wah, ba)


def kernel(x, edge_attr, W1, b1, Wm1, bm1, Wm2, bm2, Wm3, bm3, Wa, ba,
           edge_index):
    pad = jnp.zeros((NCPAD * CH - E,), jnp.int32)
    srcp = jnp.concatenate([edge_index[0].astype(jnp.int32), pad]).reshape(NCPAD, CH)
    dstp = jnp.concatenate([edge_index[1].astype(jnp.int32), pad]).reshape(NCPAD, CH)
    zeros = jnp.zeros((NP, D), jnp.float32)

    q = _tc_matmul(x, W1[:D])
    gq = _sc_gather(q, srcp)
    h = _tc_lin1(edge_attr, gq, W1[D:], b1.reshape(1, D))

    for w, b in ((Wm1, bm1), (Wm2, bm2), (Wm3, bm3)):
        parts = _sc_scatter_add(h, dstp, zeros)
        agg = _tc_sum2(parts)
        gd = _sc_gather(agg, dstp)
        h = _tc_combine(h, gd, w, b.reshape(1, D))

    parts = _sc_scatter_add(h, dstp, zeros)
    return _tc_final(x, parts, Wa[:D], Wa[D:], ba.reshape(1, D))


# speedup vs baseline: 1.5729x; 1.0000x over previous
import functools

import jax
import jax.numpy as jnp
from jax import lax
from jax.experimental import pallas as pl
from jax.experimental.pallas import tpu as pltpu
from jax.experimental.pallas import tpu_sc as plsc

N = 10000
EH = 160000
E = 2 * EH
D = 128

NC = 2
NS = 16
NW = NC * NS
CH = 128
NCHUNK = E // CH
CPW = 80
NCPAD = NW * CPW
NBUF = 6
SNBUF = 2
NP = 10240
RPT = NP // NS

_mesh = plsc.VectorSubcoreMesh(core_axis_name="c", subcore_axis_name="s")


def _worker_span():
    c = lax.axis_index("c")
    s = lax.axis_index("s")
    wid = s * NC + c
    base = wid * CPW
    cnt = jnp.clip(NCHUNK - base, 0, CPW)
    return base, cnt


def _sc_scatter_add(h2, idx2, zeros):
    @functools.partial(
        pl.kernel,
        out_type=jax.ShapeDtypeStruct((NC, NP, D), jnp.float32),
        mesh=_mesh,
        scratch_types=[
            pltpu.VMEM((CPW, CH), jnp.int32),
            pltpu.VMEM((SNBUF, CH, D), jnp.float32),
            pltpu.VMEM_SHARED((NP, D), jnp.float32),
        ] + [pltpu.SemaphoreType.DMA] * SNBUF,
    )
    def k(h_hbm, idx_hbm, z_hbm, out_hbm, idx_v, rows_v, acc, *sems):
        s = lax.axis_index("s")
        base, cnt = _worker_span()
        pltpu.sync_copy(z_hbm.at[pl.ds(s * RPT, RPT)], acc.at[pl.ds(s * RPT, RPT)])
        pltpu.sync_copy(idx_hbm.at[pl.ds(base, CPW)], idx_v)
        plsc.subcore_barrier()

        def load(i, b):
            return pltpu.make_async_copy(
                h_hbm.at[pl.ds((base + i) * CH, CH)], rows_v.at[b], sems[b])

        def drain(b):
            pltpu.make_async_copy(
                h_hbm.at[pl.ds(0, CH)], rows_v.at[b], sems[b]).wait()

        for b in range(SNBUF):
            @pl.when(b < cnt)
            def _(b=b):
                load(b, b).start()

        @pl.loop(0, CPW, step=SNBUF)
        def _(i0):
            for b in range(SNBUF):
                i = i0 + b

                @pl.when(i < cnt)
                def _(i=i, b=b):
                    drain(b)
                    pltpu.sync_copy(rows_v.at[b], acc.at[idx_v.at[i]], add=True)

                    @pl.when(i + SNBUF < cnt)
                    def _():
                        load(i + SNBUF, b).start()

        plsc.subcore_barrier()
        c = lax.axis_index("c")
        pltpu.sync_copy(acc.at[pl.ds(s * RPT, RPT)],
                        out_hbm.at[c, pl.ds(s * RPT, RPT)])

    return k(h2.reshape(E, D), idx2, zeros)


def _sc_gather(table, idx2):
    @functools.partial(
        pl.kernel,
        out_type=jax.ShapeDtypeStruct((E, D), jnp.float32),
        mesh=_mesh,
        scratch_types=[
            pltpu.VMEM((CPW, CH), jnp.int32),
            pltpu.VMEM((NBUF, CH, D), jnp.float32),
        ] + [pltpu.SemaphoreType.DMA] * NBUF,
    )
    def k(t_hbm, idx_hbm, out_hbm, idx_v, rows_v, *sems):
        base, cnt = _worker_span()
        pltpu.sync_copy(idx_hbm.at[pl.ds(base, CPW)], idx_v)

        def gat(i, b):
            return pltpu.make_async_copy(
                t_hbm.at[idx_v.at[i]], rows_v.at[b], sems[b])

        def drain(b):
            pltpu.make_async_copy(
                t_hbm.at[pl.ds(0, CH)], rows_v.at[b], sems[b]).wait()

        for b in range(NBUF):
            @pl.when(b < cnt)
            def _(b=b):
                gat(b, b).start()

        @pl.loop(0, CPW, step=NBUF)
        def _(i0):
            for b in range(NBUF):
                i = i0 + b

                @pl.when(i < cnt)
                def _(i=i, b=b):
                    drain(b)
                    pltpu.sync_copy(rows_v.at[b],
                                    out_hbm.at[pl.ds((base + i) * CH, CH)])

                    @pl.when(i + NBUF < cnt)
                    def _():
                        gat(i + NBUF, b).start()

    return k(table, idx2)


def _matmul_body(x_ref, w_ref, o_ref):
    o_ref[...] = jnp.dot(x_ref[...], w_ref[...],
                         preferred_element_type=jnp.float32
                         ).astype(o_ref.dtype)


def _tc_matmul(x, w, bm=2000, out_dtype=jnp.float32):
    m, kdim = x.shape
    dout = w.shape[1]
    return pl.pallas_call(
        _matmul_body,
        grid=(m // bm,),
        in_specs=[
            pl.BlockSpec((bm, kdim), lambda i: (i, 0)),
            pl.BlockSpec((kdim, dout), lambda i: (0, 0)),
        ],
        out_specs=pl.BlockSpec((bm, dout), lambda i: (i, 0)),
        out_shape=jax.ShapeDtypeStruct((m, dout), out_dtype),
    )(x, w)


def _lin1_body(ea_ref, g_ref, w_ref, b_ref, o_ref):
    o_ref[...] = jnp.maximum(
        g_ref[...].astype(jnp.float32)
        + jnp.dot(ea_ref[...], w_ref[...], preferred_element_type=jnp.float32)
        + b_ref[...], 0.0)


def _tc_lin1(ea, gq, w1e, b1, bp=2000):
    de = ea.shape[-1]
    return pl.pallas_call(
        _lin1_body,
        grid=(E // bp,),
        in_specs=[
            pl.BlockSpec((bp, de), lambda i: (i, 0)),
            pl.BlockSpec((bp, D), lambda i: (i, 0)),
            pl.BlockSpec((de, D), lambda i: (0, 0)),
            pl.BlockSpec((1, D), lambda i: (0, 0)),
        ],
        out_specs=pl.BlockSpec((bp, D), lambda i: (i, 0)),
        out_shape=jax.ShapeDtypeStruct((E, D), jnp.float32),
    )(ea, gq, w1e, b1)


def _combine_body(h_ref, g_ref, w_ref, b_ref, o_ref):
    hb = h_ref[...]
    z = jnp.dot(g_ref[...].astype(jnp.float32) - hb, w_ref[...],
                preferred_element_type=jnp.float32)
    even = (lax.broadcasted_iota(jnp.int32, z.shape, 0) % 2) == 0
    zsw = jnp.where(even, jnp.roll(z, -1, axis=0), jnp.roll(z, 1, axis=0))
    o_ref[...] = jnp.maximum(hb + zsw + b_ref[...], 0.0)


def _tc_combine(h2, g2, w, b, bp=2000):
    return pl.pallas_call(
        _combine_body,
        grid=(E // bp,),
        in_specs=[
            pl.BlockSpec((bp, D), lambda i: (i, 0)),
            pl.BlockSpec((bp, D), lambda i: (i, 0)),
            pl.BlockSpec((D, D), lambda i: (0, 0)),
            pl.BlockSpec((1, D), lambda i: (0, 0)),
        ],
        out_specs=pl.BlockSpec((bp, D), lambda i: (i, 0)),
        out_shape=jax.ShapeDtypeStruct((E, D), jnp.float32),
    )(h2, g2, w, b)


def _sum2_body(p_ref, o_ref):
    o_ref[...] = (p_ref[0] + p_ref[1]).astype(o_ref.dtype)


def _tc_sum2(parts, bn=2000):
    return pl.pallas_call(
        _sum2_body,
        grid=(N // bn,),
        in_specs=[pl.BlockSpec((2, bn, D), lambda i: (0, i, 0))],
        out_specs=pl.BlockSpec((bn, D), lambda i: (i, 0)),
        out_shape=jax.ShapeDtypeStruct((N, D), jnp.float32),
    )(parts)


def _final_body(x_ref, p_ref, wx_ref, wh_ref, b_ref, o_ref):
    agg = p_ref[0] + p_ref[1]
    o_ref[...] = jnp.maximum(
        jnp.dot(x_ref[...], wx_ref[...], preferred_element_type=jnp.float32)
        + jnp.dot(agg, wh_ref[...], preferred_element_type=jnp.float32)
        + b_ref[...], 0.0)


def _tc_final(x, parts, wax, wah, ba, bn=2000):
    return pl.pallas_call(
        _final_body,
        grid=(N // bn,),
        in_specs=[
            pl.BlockSpec((bn, D), lambda i: (i, 0)),
            pl.BlockSpec((2, bn, D), lambda i: (0, i, 0)),
            pl.BlockSpec((D, D), lambda i: (0, 0)),
            pl.BlockSpec((D, D), lambda i: (0, 0)),
            pl.BlockSpec((1, D), lambda i: (0, 0)),
        ],
        out_specs=pl.BlockSpec((bn, D), lambda i: (i, 0)),
        out_shape=jax.ShapeDtypeStruct((N, D), jnp.float32),
    )(x, parts, wax, wah, ba)


def kernel(x, edge_attr, W1, b1, Wm1, bm1, Wm2, bm2, Wm3, bm3, Wa, ba,
           edge_index):
    pad = jnp.zeros((NCPAD * CH - E,), jnp.int32)
    srcp = jnp.concatenate([edge_index[0].astype(jnp.int32), pad]).reshape(NCPAD, CH)
    dstp = jnp.concatenate([edge_index[1].astype(jnp.int32), pad]).reshape(NCPAD, CH)
    zeros = jnp.zeros((NP, D), jnp.float32)

    q = _tc_matmul(x, W1[:D])
    gq = _sc_gather(q, srcp)
    h = _tc_lin1(edge_attr, gq, W1[D:], b1.reshape(1, D))

    for w, b in ((Wm1, bm1), (Wm2, bm2), (Wm3, bm3)):
        parts = _sc_scatter_add(h, dstp, zeros)
        agg = _tc_sum2(parts)
        gd = _sc_gather(agg, dstp)
        h = _tc_combine(h, gd, w, b.reshape(1, D))

    parts = _sc_scatter_add(h, dstp, zeros)
    return _tc_final(x, parts, Wa[:D], Wa[D:], ba.reshape(1, D))


# speedup vs baseline: 1.6789x; 1.0674x over previous
import functools

import jax
import jax.numpy as jnp
from jax import lax
from jax.experimental import pallas as pl
from jax.experimental.pallas import tpu as pltpu
from jax.experimental.pallas import tpu_sc as plsc

N = 10000
E = 320000
EHALF = E // 2
D = 128

NC = 2
NS = 16
NW = NC * NS
CH = 128
NCHUNK = EHALF // CH
CPW = 40
NCPAD = NW * CPW
NBUF = 6
SNBUF = 2
NP = 10240
RPT = NP // NS

_mesh = plsc.VectorSubcoreMesh(core_axis_name="c", subcore_axis_name="s")


def _worker_span():
    c = lax.axis_index("c")
    s = lax.axis_index("s")
    wid = s * NC + c
    base = wid * CPW
    cnt = jnp.clip(NCHUNK - base, 0, CPW)
    return base, cnt


def _sc_scatter_add(hh, idx2, zeros):
    @functools.partial(
        pl.kernel,
        out_type=jax.ShapeDtypeStruct((NC, NP, D), jnp.float32),
        mesh=_mesh,
        scratch_types=[
            pltpu.VMEM((CPW, CH), jnp.int32),
            pltpu.VMEM((SNBUF, CH, D), jnp.float32),
            pltpu.VMEM_SHARED((NP, D), jnp.float32),
        ] + [pltpu.SemaphoreType.DMA] * SNBUF,
    )
    def k(h_hbm, idx_hbm, z_hbm, out_hbm, idx_v, rows_v, acc, *sems):
        s = lax.axis_index("s")
        base, cnt = _worker_span()
        pltpu.sync_copy(z_hbm.at[pl.ds(s * RPT, RPT)], acc.at[pl.ds(s * RPT, RPT)])
        pltpu.sync_copy(idx_hbm.at[pl.ds(base, CPW)], idx_v)
        plsc.subcore_barrier()

        def load(i, b):
            return pltpu.make_async_copy(
                h_hbm.at[pl.ds((base + i) * CH, CH)], rows_v.at[b], sems[b])

        def drain(b):
            pltpu.make_async_copy(
                h_hbm.at[pl.ds(0, CH)], rows_v.at[b], sems[b]).wait()

        for b in range(SNBUF):
            @pl.when(b < cnt)
            def _(b=b):
                load(b, b).start()

        @pl.loop(0, CPW, step=SNBUF)
        def _(i0):
            for b in range(SNBUF):
                i = i0 + b

                @pl.when(i < cnt)
                def _(i=i, b=b):
                    drain(b)
                    pltpu.sync_copy(rows_v.at[b], acc.at[idx_v.at[i]], add=True)

                    @pl.when(i + SNBUF < cnt)
                    def _():
                        load(i + SNBUF, b).start()

        plsc.subcore_barrier()
        c = lax.axis_index("c")
        pltpu.sync_copy(acc.at[pl.ds(s * RPT, RPT)],
                        out_hbm.at[c, pl.ds(s * RPT, RPT)])

    return k(hh, idx2, zeros)


def _sc_gather(table, idx2):
    @functools.partial(
        pl.kernel,
        out_type=jax.ShapeDtypeStruct((EHALF, D), jnp.float32),
        mesh=_mesh,
        scratch_types=[
            pltpu.VMEM((CPW, CH), jnp.int32),
            pltpu.VMEM((NBUF, CH, D), jnp.float32),
        ] + [pltpu.SemaphoreType.DMA] * NBUF,
    )
    def k(t_hbm, idx_hbm, out_hbm, idx_v, rows_v, *sems):
        base, cnt = _worker_span()
        pltpu.sync_copy(idx_hbm.at[pl.ds(base, CPW)], idx_v)

        def gat(i, b):
            return pltpu.make_async_copy(
                t_hbm.at[idx_v.at[i]], rows_v.at[b], sems[b])

        def drain(b):
            pltpu.make_async_copy(
                t_hbm.at[pl.ds(0, CH)], rows_v.at[b], sems[b]).wait()

        for b in range(NBUF):
            @pl.when(b < cnt)
            def _(b=b):
                gat(b, b).start()

        @pl.loop(0, CPW, step=NBUF)
        def _(i0):
            for b in range(NBUF):
                i = i0 + b

                @pl.when(i < cnt)
                def _(i=i, b=b):
                    drain(b)
                    pltpu.sync_copy(rows_v.at[b],
                                    out_hbm.at[pl.ds((base + i) * CH, CH)])

                    @pl.when(i + NBUF < cnt)
                    def _():
                        gat(i + NBUF, b).start()

    return k(table, idx2)


def _matmul_body(x_ref, w_ref, o_ref):
    o_ref[...] = jnp.dot(x_ref[...], w_ref[...],
                         preferred_element_type=jnp.float32)


def _tc_matmul(x, w, bm=2000):
    m, kdim = x.shape
    dout = w.shape[1]
    return pl.pallas_call(
        _matmul_body,
        grid=(m // bm,),
        in_specs=[
            pl.BlockSpec((bm, kdim), lambda i: (i, 0)),
            pl.BlockSpec((kdim, dout), lambda i: (0, 0)),
        ],
        out_specs=pl.BlockSpec((bm, dout), lambda i: (i, 0)),
        out_shape=jax.ShapeDtypeStruct((m, dout), jnp.float32),
    )(x, w)


def _lin1_body(ea_ref, g_ref, w_ref, b_ref, o_ref):
    o_ref[...] = jnp.maximum(
        g_ref[...]
        + jnp.dot(ea_ref[...], w_ref[...], preferred_element_type=jnp.float32)
        + b_ref[...], 0.0)


def _tc_lin1(ea, gq, w1e, b1, bp=2000):
    de = ea.shape[-1]
    return pl.pallas_call(
        _lin1_body,
        grid=(EHALF // bp,),
        in_specs=[
            pl.BlockSpec((bp, de), lambda i: (i, 0)),
            pl.BlockSpec((bp, D), lambda i: (i, 0)),
            pl.BlockSpec((de, D), lambda i: (0, 0)),
            pl.BlockSpec((1, D), lambda i: (0, 0)),
        ],
        out_specs=pl.BlockSpec((bp, D), lambda i: (i, 0)),
        out_shape=jax.ShapeDtypeStruct((EHALF, D), jnp.float32),
    )(ea, gq, w1e, b1)


def _combine_body(h_ref, g_ref, w_ref, b_ref, o_ref):
    hb = h_ref[...]
    z = jnp.dot(g_ref[...] - hb, w_ref[...],
                preferred_element_type=jnp.float32)
    even = (lax.broadcasted_iota(jnp.int32, z.shape, 0) % 2) == 0
    zsw = jnp.where(even, jnp.roll(z, -1, axis=0), jnp.roll(z, 1, axis=0))
    o_ref[...] = jnp.maximum(hb + zsw + b_ref[...], 0.0)


def _tc_combine(hh, gh, w, b, bp=2000):
    return pl.pallas_call(
        _combine_body,
        grid=(EHALF // bp,),
        in_specs=[
            pl.BlockSpec((bp, D), lambda i: (i, 0)),
            pl.BlockSpec((bp, D), lambda i: (i, 0)),
            pl.BlockSpec((D, D), lambda i: (0, 0)),
            pl.BlockSpec((1, D), lambda i: (0, 0)),
        ],
        out_specs=pl.BlockSpec((bp, D), lambda i: (i, 0)),
        out_shape=jax.ShapeDtypeStruct((EHALF, D), jnp.float32),
    )(hh, gh, w, b)


def _sum4_body(pa_ref, pb_ref, o_ref):
    o_ref[...] = (pa_ref[0] + pa_ref[1]) + (pb_ref[0] + pb_ref[1])


def _tc_sum4(pa, pb, bn=2000):
    return pl.pallas_call(
        _sum4_body,
        grid=(N // bn,),
        in_specs=[
            pl.BlockSpec((2, bn, D), lambda i: (0, i, 0)),
            pl.BlockSpec((2, bn, D), lambda i: (0, i, 0)),
        ],
        out_specs=pl.BlockSpec((bn, D), lambda i: (i, 0)),
        out_shape=jax.ShapeDtypeStruct((N, D), jnp.float32),
    )(pa, pb)


def _final_body(x_ref, pa_ref, pb_ref, wx_ref, wh_ref, b_ref, o_ref):
    agg = (pa_ref[0] + pa_ref[1]) + (pb_ref[0] + pb_ref[1])
    o_ref[...] = jnp.maximum(
        jnp.dot(x_ref[...], wx_ref[...], preferred_element_type=jnp.float32)
        + jnp.dot(agg, wh_ref[...], preferred_element_type=jnp.float32)
        + b_ref[...], 0.0)


def _tc_final(x, pa, pb, wax, wah, ba, bn=2000):
    return pl.pallas_call(
        _final_body,
        grid=(N // bn,),
        in_specs=[
            pl.BlockSpec((bn, D), lambda i: (i, 0)),
            pl.BlockSpec((2, bn, D), lambda i: (0, i, 0)),
            pl.BlockSpec((2, bn, D), lambda i: (0, i, 0)),
            pl.BlockSpec((D, D), lambda i: (0, 0)),
            pl.BlockSpec((D, D), lambda i: (0, 0)),
            pl.BlockSpec((1, D), lambda i: (0, 0)),
        ],
        out_specs=pl.BlockSpec((bn, D), lambda i: (i, 0)),
        out_shape=jax.ShapeDtypeStruct((N, D), jnp.float32),
    )(x, pa, pb, wax, wah, ba)


def kernel(x, edge_attr, W1, b1, Wm1, bm1, Wm2, bm2, Wm3, bm3, Wa, ba,
           edge_index):
    pad = jnp.zeros((NCPAD * CH - EHALF,), jnp.int32)
    src = edge_index[0].astype(jnp.int32)
    dst = edge_index[1].astype(jnp.int32)
    srcA = jnp.concatenate([src[:EHALF], pad]).reshape(NCPAD, CH)
    srcB = jnp.concatenate([src[EHALF:], pad]).reshape(NCPAD, CH)
    dstA = jnp.concatenate([dst[:EHALF], pad]).reshape(NCPAD, CH)
    dstB = jnp.concatenate([dst[EHALF:], pad]).reshape(NCPAD, CH)
    zeros = jnp.zeros((NP, D), jnp.float32)
    b1r = b1.reshape(1, D)

    q = _tc_matmul(x, W1[:D])
    gqA = _sc_gather(q, srcA)
    gqB = _sc_gather(q, srcB)
    hA = _tc_lin1(edge_attr[:EHALF], gqA, W1[D:], b1r)
    hB = _tc_lin1(edge_attr[EHALF:], gqB, W1[D:], b1r)

    for w, b in ((Wm1, bm1), (Wm2, bm2), (Wm3, bm3)):
        pA = _sc_scatter_add(hA, dstA, zeros)
        pB = _sc_scatter_add(hB, dstB, zeros)
        agg = _tc_sum4(pA, pB)
        gdA = _sc_gather(agg, dstA)
        gdB = _sc_gather(agg, dstB)
        br = b.reshape(1, D)
        hA = _tc_combine(hA, gdA, w, br)
        hB = _tc_combine(hB, gdB, w, br)

    pA = _sc_scatter_add(hA, dstA, zeros)
    pB = _sc_scatter_add(hB, dstB, zeros)
    return _tc_final(x, pA, pB, Wa[:D], Wa[D:], ba.reshape(1, D))


# speedup vs baseline: 1.7009x; 1.0131x over previous
import functools

import jax
import jax.numpy as jnp
from jax import lax
from jax.experimental import pallas as pl
from jax.experimental.pallas import tpu as pltpu
from jax.experimental.pallas import tpu_sc as plsc

N = 10000
E = 320000
NSPLIT = 4
EPC = E // NSPLIT
D = 128

NC = 2
NS = 16
NW = NC * NS
CH = 128
NCHUNK = EPC // CH
CPW = 20
NCPAD = NW * CPW
NBUF = 6
SNBUF = 2
NP = 10240
RPT = NP // NS

_mesh = plsc.VectorSubcoreMesh(core_axis_name="c", subcore_axis_name="s")


def _worker_span():
    c = lax.axis_index("c")
    s = lax.axis_index("s")
    wid = s * NC + c
    base = wid * CPW
    cnt = jnp.clip(NCHUNK - base, 0, CPW)
    return wid, base, cnt


def _sc_scatter_add(hh, idx2, zeros):
    @functools.partial(
        pl.kernel,
        out_type=jax.ShapeDtypeStruct((NC, NP, D), jnp.float32),
        mesh=_mesh,
        scratch_types=[
            pltpu.VMEM((CPW, CH), jnp.int32),
            pltpu.VMEM((SNBUF, CH, D), jnp.float32),
            pltpu.VMEM_SHARED((NP, D), jnp.float32),
        ] + [pltpu.SemaphoreType.DMA] * SNBUF,
    )
    def k(h_hbm, idx_hbm, z_hbm, out_hbm, idx_v, rows_v, acc, *sems):
        s = lax.axis_index("s")
        wid, base, cnt = _worker_span()
        pltpu.sync_copy(z_hbm.at[pl.ds(s * RPT, RPT)], acc.at[pl.ds(s * RPT, RPT)])
        pltpu.sync_copy(idx_hbm.at[wid], idx_v)
        plsc.subcore_barrier()

        def load(i, b):
            return pltpu.make_async_copy(
                h_hbm.at[pl.ds((base + i) * CH, CH)], rows_v.at[b], sems[b])

        def drain(b):
            pltpu.make_async_copy(
                h_hbm.at[pl.ds(0, CH)], rows_v.at[b], sems[b]).wait()

        for b in range(SNBUF):
            @pl.when(b < cnt)
            def _(b=b):
                load(b, b).start()

        @pl.loop(0, CPW, step=SNBUF)
        def _(i0):
            for b in range(SNBUF):
                i = i0 + b

                @pl.when(i < cnt)
                def _(i=i, b=b):
                    drain(b)
                    pltpu.sync_copy(rows_v.at[b], acc.at[idx_v.at[i]], add=True)

                    @pl.when(i + SNBUF < cnt)
                    def _():
                        load(i + SNBUF, b).start()

        plsc.subcore_barrier()
        c = lax.axis_index("c")
        pltpu.sync_copy(acc.at[pl.ds(s * RPT, RPT)],
                        out_hbm.at[c, pl.ds(s * RPT, RPT)])

    return k(hh, idx2, zeros)


def _sc_gather(table, idx2):
    @functools.partial(
        pl.kernel,
        out_type=jax.ShapeDtypeStruct((EPC, D), jnp.float32),
        mesh=_mesh,
        scratch_types=[
            pltpu.VMEM((CPW, CH), jnp.int32),
            pltpu.VMEM((NBUF, CH, D), jnp.float32),
        ] + [pltpu.SemaphoreType.DMA] * NBUF,
    )
    def k(t_hbm, idx_hbm, out_hbm, idx_v, rows_v, *sems):
        wid, base, cnt = _worker_span()
        pltpu.sync_copy(idx_hbm.at[wid], idx_v)

        def gat(i, b):
            return pltpu.make_async_copy(
                t_hbm.at[idx_v.at[i]], rows_v.at[b], sems[b])

        def drain(b):
            pltpu.make_async_copy(
                t_hbm.at[pl.ds(0, CH)], rows_v.at[b], sems[b]).wait()

        for b in range(NBUF):
            @pl.when(b < cnt)
            def _(b=b):
                gat(b, b).start()

        @pl.loop(0, CPW, step=NBUF)
        def _(i0):
            for b in range(NBUF):
                i = i0 + b

                @pl.when(i < cnt)
                def _(i=i, b=b):
                    drain(b)
                    pltpu.sync_copy(rows_v.at[b],
                                    out_hbm.at[pl.ds((base + i) * CH, CH)])

                    @pl.when(i + NBUF < cnt)
                    def _():
                        gat(i + NBUF, b).start()

    return k(table, idx2)


def _matmul_body(x_ref, w_ref, o_ref):
    o_ref[...] = jnp.dot(x_ref[...], w_ref[...],
                         preferred_element_type=jnp.float32)


def _tc_matmul(x, w, bm=2000):
    m, kdim = x.shape
    dout = w.shape[1]
    return pl.pallas_call(
        _matmul_body,
        grid=(m // bm,),
        in_specs=[
            pl.BlockSpec((bm, kdim), lambda i: (i, 0)),
            pl.BlockSpec((kdim, dout), lambda i: (0, 0)),
        ],
        out_specs=pl.BlockSpec((bm, dout), lambda i: (i, 0)),
        out_shape=jax.ShapeDtypeStruct((m, dout), jnp.float32),
    )(x, w)


def _lin1_body(ea_ref, g_ref, w_ref, b_ref, o_ref):
    o_ref[...] = jnp.maximum(
        g_ref[...]
        + jnp.dot(ea_ref[...], w_ref[...], preferred_element_type=jnp.float32)
        + b_ref[...], 0.0)


def _tc_lin1(ea, gq, w1e, b1, bp=2000):
    de = ea.shape[-1]
    return pl.pallas_call(
        _lin1_body,
        grid=(EPC // bp,),
        in_specs=[
            pl.BlockSpec((bp, de), lambda i: (i, 0)),
            pl.BlockSpec((bp, D), lambda i: (i, 0)),
            pl.BlockSpec((de, D), lambda i: (0, 0)),
            pl.BlockSpec((1, D), lambda i: (0, 0)),
        ],
        out_specs=pl.BlockSpec((bp, D), lambda i: (i, 0)),
        out_shape=jax.ShapeDtypeStruct((EPC, D), jnp.float32),
    )(ea, gq, w1e, b1)


def _combine_body(h_ref, g_ref, w_ref, b_ref, o_ref):
    hb = h_ref[...]
    z = jnp.dot(g_ref[...] - hb, w_ref[...],
                preferred_element_type=jnp.float32)
    even = (lax.broadcasted_iota(jnp.int32, z.shape, 0) % 2) == 0
    zsw = jnp.where(even, jnp.roll(z, -1, axis=0), jnp.roll(z, 1, axis=0))
    o_ref[...] = jnp.maximum(hb + zsw + b_ref[...], 0.0)


def _tc_combine(hh, gh, w, b, bp=2000):
    return pl.pallas_call(
        _combine_body,
        grid=(EPC // bp,),
        in_specs=[
            pl.BlockSpec((bp, D), lambda i: (i, 0)),
            pl.BlockSpec((bp, D), lambda i: (i, 0)),
            pl.BlockSpec((D, D), lambda i: (0, 0)),
            pl.BlockSpec((1, D), lambda i: (0, 0)),
        ],
        out_specs=pl.BlockSpec((bp, D), lambda i: (i, 0)),
        out_shape=jax.ShapeDtypeStruct((EPC, D), jnp.float32),
    )(hh, gh, w, b)


def _sum_parts_body(*refs):
    o_ref = refs[-1]
    acc = refs[0][0] + refs[0][1]
    for r in refs[1:-1]:
        acc = acc + (r[0] + r[1])
    o_ref[...] = acc


def _tc_sum_parts(parts, bn=2000):
    return pl.pallas_call(
        _sum_parts_body,
        grid=(N // bn,),
        in_specs=[pl.BlockSpec((2, bn, D), lambda i: (0, i, 0))
                  for _ in parts],
        out_specs=pl.BlockSpec((bn, D), lambda i: (i, 0)),
        out_shape=jax.ShapeDtypeStruct((N, D), jnp.float32),
    )(*parts)


def _final_body(*refs):
    x_ref = refs[0]
    parts = refs[1:1 + NSPLIT]
    wx_ref, wh_ref, b_ref, o_ref = refs[1 + NSPLIT:]
    agg = parts[0][0] + parts[0][1]
    for r in parts[1:]:
        agg = agg + (r[0] + r[1])
    o_ref[...] = jnp.maximum(
        jnp.dot(x_ref[...], wx_ref[...], preferred_element_type=jnp.float32)
        + jnp.dot(agg, wh_ref[...], preferred_element_type=jnp.float32)
        + b_ref[...], 0.0)


def _tc_final(x, parts, wax, wah, ba, bn=2000):
    return pl.pallas_call(
        _final_body,
        grid=(N // bn,),
        in_specs=[pl.BlockSpec((bn, D), lambda i: (i, 0))]
        + [pl.BlockSpec((2, bn, D), lambda i: (0, i, 0)) for _ in parts]
        + [
            pl.BlockSpec((D, D), lambda i: (0, 0)),
            pl.BlockSpec((D, D), lambda i: (0, 0)),
            pl.BlockSpec((1, D), lambda i: (0, 0)),
        ],
        out_specs=pl.BlockSpec((bn, D), lambda i: (i, 0)),
        out_shape=jax.ShapeDtypeStruct((N, D), jnp.float32),
    )(x, *parts, wax, wah, ba)


def kernel(x, edge_attr, W1, b1, Wm1, bm1, Wm2, bm2, Wm3, bm3, Wa, ba,
           edge_index):
    pad = jnp.zeros((NCPAD * CH - EPC,), jnp.int32)
    srci = edge_index[0].astype(jnp.int32)
    dsti = edge_index[1].astype(jnp.int32)
    srcp = [jnp.concatenate([srci[j * EPC:(j + 1) * EPC], pad]).reshape(NW, CPW, CH)
            for j in range(NSPLIT)]
    dstp = [jnp.concatenate([dsti[j * EPC:(j + 1) * EPC], pad]).reshape(NW, CPW, CH)
            for j in range(NSPLIT)]
    zeros = jnp.zeros((NP, D), jnp.float32)
    b1r = b1.reshape(1, D)

    q = _tc_matmul(x, W1[:D])
    gq = [_sc_gather(q, srcp[j]) for j in range(NSPLIT)]
    h = [_tc_lin1(edge_attr[j * EPC:(j + 1) * EPC], gq[j], W1[D:], b1r)
         for j in range(NSPLIT)]

    for w, b in ((Wm1, bm1), (Wm2, bm2), (Wm3, bm3)):
        parts = [_sc_scatter_add(h[j], dstp[j], zeros) for j in range(NSPLIT)]
        agg = _tc_sum_parts(parts)
        br = b.reshape(1, D)
        for j in range(NSPLIT):
            gd = _sc_gather(agg, dstp[j])
            h[j] = _tc_combine(h[j], gd, w, br)

    parts = [_sc_scatter_add(h[j], dstp[j], zeros) for j in range(NSPLIT)]
    return _tc_final(x, parts, Wa[:D], Wa[D:], ba.reshape(1, D))
